# baseline (speedup 1.0000x reference)
import functools

import jax
import jax.numpy as jnp
from jax import lax
from jax.experimental import pallas as pl
from jax.experimental.pallas import tpu as pltpu
from jax.experimental.pallas import tpu_sc as plsc

NUM_CORES = 2
NUM_SUBCORES = 16
LANES = 16

WINDOWS = 4096
TOPK = 2
D = 1024
EXPERTS = 8
ROWS = WINDOWS * TOPK
OUT_ROWS = EXPERTS * WINDOWS

WIN_PER_TILE = WINDOWS // (NUM_CORES * NUM_SUBCORES)
ROWS_PER_TILE = WIN_PER_TILE * TOPK
CHUNK = 32
N_CHUNKS = ROWS_PER_TILE // CHUNK
NBUF = 3
ZFIRES = WIN_PER_TILE * (EXPERTS - TOPK)


def _dispatch_body(x_hbm, idx_hbm, out_hbm,
                   xbuf0, xbuf1, xbuf2, idxv, zrow2,
                   dst0, dst1, dst2, dst3, dst4, dst5, dst6, dst7,
                   zsem, gsem0, gsem1, gsem2, ssem0, ssem1, ssem2):
    c = lax.axis_index("c")
    s = lax.axis_index("s")
    w0 = (c * NUM_SUBCORES + s) * WIN_PER_TILE
    row0 = w0 * TOPK

    bufs = (xbuf0, xbuf1, xbuf2)
    gsems = (gsem0, gsem1, gsem2)
    ssems = (ssem0, ssem1, ssem2)
    dsts = (dst0, dst1, dst2, dst3, dst4, dst5, dst6, dst7)

    g = [None] * NBUF
    for k in range(NBUF - 1):
        g[k] = pltpu.async_copy(
            x_hbm.at[pl.ds(row0 + k * CHUNK, CHUNK)], bufs[k], gsems[k])
    pltpu.sync_copy(idx_hbm.at[pl.ds(row0, ROWS_PER_TILE)],
                    idxv.at[pl.ds(0, ROWS_PER_TILE)])

    lane = lax.iota(jnp.int32, 16)
    for k in range(N_CHUNKS):
        for i in range(CHUNK // LANES):
            r = row0 + k * CHUNK + i * LANES + lane
            e = idxv[pl.ds(k * CHUNK + i * LANES, LANES)]
            dsts[k][pl.ds(i * LANES, LANES)] = e * WINDOWS + (r >> 1)

    zero16 = jnp.zeros((LANES,), jnp.float32)

    @pl.loop(0, 2)
    def _zr(j):
        @pl.loop(0, D // LANES)
        def _zseg(i):
            zrow2[j, pl.ds(i * LANES, LANES)] = zero16

    @pl.loop(0, WIN_PER_TILE // 2)
    def _win(wp):
        quad = idxv[pl.ds(2 * TOPK * wp, LANES)]
        s0 = quad[0]
        s1 = quad[1]
        s2 = quad[2]
        s3 = quad[3]
        for e in range(EXPERTS):
            unc_a = jnp.logical_and(s0 != e, s1 != e)
            unc_b = jnp.logical_and(s2 != e, s3 != e)
            base = e * WINDOWS + w0 + 2 * wp

            @pl.when(jnp.logical_and(unc_a, unc_b))
            def _fire2(base=base):
                pltpu.async_copy(zrow2, out_hbm.at[pl.ds(base, 2)], zsem)

            @pl.when(jnp.logical_and(unc_a, jnp.logical_not(unc_b)))
            def _fire_a(base=base):
                pltpu.async_copy(
                    zrow2.at[pl.ds(0, 1)], out_hbm.at[pl.ds(base, 1)], zsem)

            @pl.when(jnp.logical_and(jnp.logical_not(unc_a), unc_b))
            def _fire_b(base=base):
                pltpu.async_copy(
                    zrow2.at[pl.ds(0, 1)], out_hbm.at[pl.ds(base + 1, 1)],
                    zsem)

    scat = [None] * NBUF
    for k in range(N_CHUNKS):
        q = k % NBUF
        g[q].wait()
        scat[q] = pltpu.async_copy(bufs[q], out_hbm.at[dsts[k]], ssems[q])
        nk = k + NBUF - 1
        if nk < N_CHUNKS:
            q2 = nk % NBUF
            if scat[q2] is not None:
                scat[q2].wait()
            g[q2] = pltpu.async_copy(
                x_hbm.at[pl.ds(row0 + nk * CHUNK, CHUNK)], bufs[q2], gsems[q2])

    for h in scat:
        if h is not None:
            h.wait()

    @pl.loop(0, ZFIRES // CHUNK)
    def _drain(i):
        pltpu.make_async_copy(
            out_hbm.at[pl.ds(0, CHUNK)], xbuf0, zsem).wait()


_dispatch = functools.partial(
    pl.kernel,
    out_type=jax.ShapeDtypeStruct((OUT_ROWS, D), jnp.float32),
    mesh=plsc.VectorSubcoreMesh(
        core_axis_name="c", subcore_axis_name="s",
        num_cores=NUM_CORES, num_subcores=NUM_SUBCORES),
    scratch_types=[
        pltpu.VMEM((CHUNK, D), jnp.float32),
        pltpu.VMEM((CHUNK, D), jnp.float32),
        pltpu.VMEM((CHUNK, D), jnp.float32),
        pltpu.VMEM((ROWS_PER_TILE + LANES,), jnp.int32),
        pltpu.VMEM((2, D), jnp.float32),
        pltpu.VMEM((CHUNK,), jnp.int32),
        pltpu.VMEM((CHUNK,), jnp.int32),
        pltpu.VMEM((CHUNK,), jnp.int32),
        pltpu.VMEM((CHUNK,), jnp.int32),
        pltpu.VMEM((CHUNK,), jnp.int32),
        pltpu.VMEM((CHUNK,), jnp.int32),
        pltpu.VMEM((CHUNK,), jnp.int32),
        pltpu.VMEM((CHUNK,), jnp.int32),
        pltpu.SemaphoreType.DMA,
        pltpu.SemaphoreType.DMA,
        pltpu.SemaphoreType.DMA,
        pltpu.SemaphoreType.DMA,
        pltpu.SemaphoreType.DMA,
        pltpu.SemaphoreType.DMA,
        pltpu.SemaphoreType.DMA,
    ],
)(_dispatch_body)


def kernel(isp_per_win, expert_indices, num_experts):
    batches, windows, k, embed_dim = isp_per_win.shape
    num_windows = batches * windows
    x = isp_per_win.reshape(num_windows * k, embed_dim)
    idx = expert_indices.reshape(-1)
    out = _dispatch(x, idx)
    return out.reshape(EXPERTS, num_windows, embed_dim)


# speedup vs baseline: 1.0218x; 1.0218x over previous
import functools

import jax
import jax.numpy as jnp
from jax import lax
from jax.experimental import pallas as pl
from jax.experimental.pallas import tpu as pltpu
from jax.experimental.pallas import tpu_sc as plsc

NUM_CORES = 2
NUM_SUBCORES = 16
LANES = 16

WINDOWS = 4096
TOPK = 2
D = 1024
EXPERTS = 8
ROWS = WINDOWS * TOPK
OUT_ROWS = EXPERTS * WINDOWS

WIN_PER_TILE = WINDOWS // (NUM_CORES * NUM_SUBCORES)
ROWS_PER_TILE = WIN_PER_TILE * TOPK
CHUNK = 32
N_CHUNKS = ROWS_PER_TILE // CHUNK
NBUF = 3
ZFIRES = WIN_PER_TILE * (EXPERTS - TOPK)


def _dispatch_body(x_hbm, idx_hbm, out_hbm,
                   xbuf0, xbuf1, xbuf2, idxv, zrow,
                   dst0, dst1, dst2, dst3, dst4, dst5, dst6, dst7,
                   zsem, gsem0, gsem1, gsem2, ssem0, ssem1, ssem2):
    c = lax.axis_index("c")
    s = lax.axis_index("s")
    w0 = (c * NUM_SUBCORES + s) * WIN_PER_TILE
    row0 = w0 * TOPK

    bufs = (xbuf0, xbuf1, xbuf2)
    gsems = (gsem0, gsem1, gsem2)
    ssems = (ssem0, ssem1, ssem2)
    dsts = (dst0, dst1, dst2, dst3, dst4, dst5, dst6, dst7)

    g = [None] * NBUF
    for k in range(NBUF - 1):
        g[k] = pltpu.async_copy(
            x_hbm.at[pl.ds(row0 + k * CHUNK, CHUNK)], bufs[k], gsems[k])
    pltpu.sync_copy(idx_hbm.at[pl.ds(row0, ROWS_PER_TILE)],
                    idxv.at[pl.ds(0, ROWS_PER_TILE)])

    lane = lax.iota(jnp.int32, 16)
    for k in range(N_CHUNKS):
        for i in range(CHUNK // LANES):
            r = row0 + k * CHUNK + i * LANES + lane
            e = idxv[pl.ds(k * CHUNK + i * LANES, LANES)]
            dsts[k][pl.ds(i * LANES, LANES)] = e * WINDOWS + (r >> 1)

    zero16 = jnp.zeros((LANES,), jnp.float32)

    @pl.loop(0, D // LANES)
    def _zseg(i):
        zrow[0, pl.ds(i * LANES, LANES)] = zero16

    for e in range(EXPERTS):
        @pl.loop(0, WIN_PER_TILE)
        def _win(w, e=e):
            pair = idxv[pl.ds(TOPK * w, LANES)]
            s0 = pair[0]
            s1 = pair[1]

            @pl.when(jnp.logical_and(s0 != e, s1 != e))
            def _fire(e=e, w=w):
                pltpu.async_copy(
                    zrow, out_hbm.at[pl.ds(e * WINDOWS + w0 + w, 1)], zsem)

    scat = [None] * NBUF
    for k in range(N_CHUNKS):
        q = k % NBUF
        g[q].wait()
        scat[q] = pltpu.async_copy(bufs[q], out_hbm.at[dsts[k]], ssems[q])
        nk = k + NBUF - 1
        if nk < N_CHUNKS:
            q2 = nk % NBUF
            if scat[q2] is not None:
                scat[q2].wait()
            g[q2] = pltpu.async_copy(
                x_hbm.at[pl.ds(row0 + nk * CHUNK, CHUNK)], bufs[q2], gsems[q2])

    for h in scat:
        if h is not None:
            h.wait()

    @pl.loop(0, ZFIRES // CHUNK)
    def _drain(i):
        pltpu.make_async_copy(
            out_hbm.at[pl.ds(0, CHUNK)], xbuf0, zsem).wait()


_dispatch = functools.partial(
    pl.kernel,
    out_type=jax.ShapeDtypeStruct((OUT_ROWS, D), jnp.float32),
    mesh=plsc.VectorSubcoreMesh(
        core_axis_name="c", subcore_axis_name="s",
        num_cores=NUM_CORES, num_subcores=NUM_SUBCORES),
    scratch_types=[
        pltpu.VMEM((CHUNK, D), jnp.float32),
        pltpu.VMEM((CHUNK, D), jnp.float32),
        pltpu.VMEM((CHUNK, D), jnp.float32),
        pltpu.VMEM((ROWS_PER_TILE + LANES,), jnp.int32),
        pltpu.VMEM((1, D), jnp.float32),
        pltpu.VMEM((CHUNK,), jnp.int32),
        pltpu.VMEM((CHUNK,), jnp.int32),
        pltpu.VMEM((CHUNK,), jnp.int32),
        pltpu.VMEM((CHUNK,), jnp.int32),
        pltpu.VMEM((CHUNK,), jnp.int32),
        pltpu.VMEM((CHUNK,), jnp.int32),
        pltpu.VMEM((CHUNK,), jnp.int32),
        pltpu.VMEM((CHUNK,), jnp.int32),
        pltpu.SemaphoreType.DMA,
        pltpu.SemaphoreType.DMA,
        pltpu.SemaphoreType.DMA,
        pltpu.SemaphoreType.DMA,
        pltpu.SemaphoreType.DMA,
        pltpu.SemaphoreType.DMA,
        pltpu.SemaphoreType.DMA,
    ],
)(_dispatch_body)


def kernel(isp_per_win, expert_indices, num_experts):
    batches, windows, k, embed_dim = isp_per_win.shape
    num_windows = batches * windows
    x = isp_per_win.reshape(num_windows * k, embed_dim)
    idx = expert_indices.reshape(-1)
    out = _dispatch(x, idx)
    return out.reshape(EXPERTS, num_windows, embed_dim)
